# Initial kernel scaffold; baseline (speedup 1.0000x reference)
import jax
import jax.numpy as jnp
from jax.experimental import pallas as pl


def kernel(inp, direction, prob):
    raise NotImplementedError("write your pallas kernel here")


# speedup vs baseline: 2.8994x; 2.8994x over previous
import functools

import jax
import jax.numpy as jnp
from jax.experimental import pallas as pl

_OFFS = {0: (-1, -1), 1: (-1, 0), 2: (-1, 1), 3: (0, -1),
         5: (0, 1), 6: (1, -1), 7: (1, 0)}


def _body(inp_ref, dir_ref, prob_ref, out_ref):
    a = inp_ref[...]
    G, H, W = a.shape
    e = jnp.where(prob_ref[...] <= 20, dir_ref[...], -1)

    up = jnp.concatenate([a[:, 1:2, :], a[:, :-1, :]], axis=1)
    dn = jnp.concatenate([a[:, 1:, :], a[:, -2:-1, :]], axis=1)

    rs = up + a + dn
    rl = jnp.concatenate([rs[:, :, 1:2], rs[:, :, :-1]], axis=2)
    rr = jnp.concatenate([rs[:, :, 1:], rs[:, :, -2:-1]], axis=2)
    avg = (rl + rs + rr) * (1.0 / 9.0)

    def colL(x):
        return jnp.concatenate([x[:, :, 1:2], x[:, :, :-1]], axis=2)

    def colR(x):
        return jnp.concatenate([x[:, :, 1:], x[:, :, -2:-1]], axis=2)

    si = {0: colR(dn), 1: dn, 2: colL(dn), 3: colR(a),
          5: colL(a), 6: colR(up), 7: up}

    fill_row = jnp.full((G, 1, W), -1, dtype=e.dtype)
    fill_col = jnp.full((G, H, 1), -1, dtype=e.dtype)
    ed = jnp.concatenate([e[:, 1:, :], fill_row], axis=1)
    eu = jnp.concatenate([fill_row, e[:, :-1, :]], axis=1)

    def colLm(x):
        return jnp.concatenate([fill_col, x[:, :, :-1]], axis=2)

    def colRm(x):
        return jnp.concatenate([x[:, :, 1:], fill_col], axis=2)

    se = {0: colRm(ed), 1: ed, 2: colLm(ed), 3: colRm(e),
          5: colLm(e), 6: colRm(eu), 7: eu}

    ii = jax.lax.broadcasted_iota(jnp.int32, (G, H, W), 1)
    jj = jax.lax.broadcasted_iota(jnp.int32, (G, H, W), 2)
    row_up, row_dn = ii >= 1, ii < H - 1
    col_l, col_r = jj >= 1, jj < W - 1
    inb = {0: row_up & col_l, 1: row_up, 2: row_up & col_r, 3: col_l,
           5: col_r, 6: row_dn & col_l, 7: row_dn}

    x = a
    for d in range(8):
        if d == 4:
            x = jnp.where(e == 4, avg, x)
            continue
        x = jnp.where(se[d] == d, si[d], x)
        x = jnp.where((e == d) & inb[d], avg, x)
    out_ref[...] = x


@functools.partial(jax.jit, static_argnames=("interpret",))
def kernel(inp, direction, prob, interpret=False):
    B, C, H, W = inp.shape
    N = B * C
    G = 8
    while N % G:
        G -= 1
    a3 = inp.reshape(N, H, W)
    d3 = direction.reshape(N, H, W)
    p3 = prob.reshape(N, H, W)
    spec = pl.BlockSpec((G, H, W), lambda i: (i, 0, 0))
    out = pl.pallas_call(
        _body,
        grid=(N // G,),
        in_specs=[spec, spec, spec],
        out_specs=spec,
        out_shape=jax.ShapeDtypeStruct((N, H, W), inp.dtype),
        interpret=interpret,
    )(a3, d3, p3)
    return out.reshape(B, C, H, W)
